# baseline (speedup 1.0000x reference)
import functools
import math

import jax
import jax.numpy as jnp
import numpy as np
from jax import lax
from jax.experimental import pallas as pl
from jax.experimental.pallas import tpu as pltpu
from jax.experimental.pallas import tpu_sc as plsc

VOCAB = 1000000
HIDDEN = 64
MAX_LEN = 512
BATCH = 4096
SEQ = 200

NC = 2
NS = 16
NW = NC * NS
LB = SEQ // 8
LANES = 16
PITCH = 133


def _make_pe_np(hidden_size=HIDDEN, max_len=MAX_LEN):
    position = np.arange(0, max_len, dtype=np.float32)[:, None]
    div_term = np.exp(
        np.arange(0, hidden_size, 2, dtype=np.float32)
        * (-math.log(10000.0) / hidden_size)
    )
    pe = np.zeros((max_len, hidden_size), dtype=np.float32)
    pe[:, 0::2] = np.sin(position * div_term)
    pe[:, 1::2] = np.cos(position * div_term)
    return pe


_PE = _make_pe_np()[:SEQ].reshape(-1)


def _sc_embed(ids_p, table2, pe):
    mesh = plsc.VectorSubcoreMesh(core_axis_name="c", subcore_axis_name="s")

    @functools.partial(
        pl.kernel,
        out_type=jax.ShapeDtypeStruct((SEQ, 8, NW, 8, 128), jnp.float32),
        mesh=mesh,
        compiler_params=pltpu.CompilerParams(
            use_tc_tiling_on_sc=True, needs_layout_passes=False
        ),
        scratch_types=(
            [pltpu.VMEM((SEQ * HIDDEN,), jnp.float32)]
            + [pltpu.VMEM((8, 128), jnp.int32)] * 2
            + [pltpu.VMEM((128,), jnp.int32)] * 4
            + [pltpu.VMEM((144,), jnp.int32)] * 4
            + [pltpu.VMEM((128, 128), jnp.float32)] * 4
            + [pltpu.VMEM((8, 8, PITCH), jnp.float32)] * 2
            + [pltpu.SemaphoreType.DMA] * 8
        ),
    )
    def k(ids_hbm, table_hbm, pe_hbm, out_hbm, pe_v, *rest):
        ibuf = rest[0:2]
        xbuf = rest[2:6]
        lobuf = rest[6:10]
        gbuf = rest[10:14]
        obuf = rest[14:16]
        isem = rest[16:18]
        gsem = rest[18:22]
        osem = rest[22:24]
        wid = lax.axis_index("s") * NC + lax.axis_index("c")

        pltpu.sync_copy(pe_hbm, pe_v)
        pltpu.sync_copy(ids_hbm.at[0, wid], ibuf[0])
        pltpu.async_copy(ids_hbm.at[1, wid], ibuf[1], isem[1])

        iota = lax.iota(jnp.int32, LANES)
        hvec = [iota + cg * LANES for cg in range(4)]
        i0c = [h // 8 for h in hvec]
        i1c = [h % 8 for h in hvec]

        def prep(islot, l_row, xslot, lslot):
            for g in range(8):
                v = islot[l_row, pl.ds(g * LANES, LANES)]
                xslot[pl.ds(g * LANES, LANES)] = v >> 1
                lslot[pl.ds(g * LANES, LANES)] = (v & 1) * HIDDEN

        prep(ibuf[0], 0, xbuf[0], lobuf[0])
        prep(ibuf[0], 1, xbuf[1], lobuf[1])
        pltpu.async_copy(table_hbm.at[xbuf[0]], gbuf[0], gsem[0])
        pltpu.async_copy(table_hbm.at[xbuf[1]], gbuf[1], gsem[1])

        def pairbody(pp, carry):
            for q in range(2):
                p = pp * 2 + q

                @pl.when(p < LB)
                def _():
                    for kq in range(8):
                        l = p * 8 + kq
                        g4 = kq % 4
                        r2 = kq % 2

                        if kq == 6:
                            @pl.when(p < LB - 1)
                            def _():
                                pltpu.make_async_copy(
                                    ids_hbm.at[0, wid], ibuf[1 - q],
                                    isem[1 - q],
                                ).wait()

                            @pl.when(p < LB - 2)
                            def _():
                                pltpu.async_copy(
                                    ids_hbm.at[p + 2, wid], ibuf[q], isem[q]
                                )

                        @pl.when(l < SEQ - 2)
                        def _():
                            n4 = (kq + 2) % 4
                            nsl = ibuf[q] if kq < 6 else ibuf[1 - q]
                            prep(nsl, (kq + 2) % 8, xbuf[n4], lobuf[n4])
                            pltpu.async_copy(
                                table_hbm.at[xbuf[n4]], gbuf[n4], gsem[n4]
                            )

                        pltpu.make_async_copy(
                            table_hbm.at[xbuf[g4]], gbuf[g4], gsem[g4]
                        ).wait()

                        @pl.when(l >= 2)
                        def _():
                            pltpu.make_async_copy(
                                obuf[r2].at[:, :, pl.ds(0, 128)],
                                out_hbm.at[0, :, wid],
                                osem[r2],
                            ).wait()

                        pev = [
                            pe_v[pl.ds(l * HIDDEN + cg * LANES, LANES)]
                            for cg in range(4)
                        ]

                        @plsc.parallel_loop(0, 128, unroll=4)
                        def rowloop(r):
                            rb = jnp.broadcast_to(r, (LANES,))
                            pv = plsc.load_gather(lobuf[g4], [rb])
                            for cg in range(4):
                                v = plsc.load_gather(
                                    gbuf[g4], [rb, pv + hvec[cg]]
                                )
                                plsc.store_scatter(
                                    obuf[r2], [i0c[cg], i1c[cg], rb],
                                    v + pev[cg],
                                )

                        pltpu.async_copy(
                            obuf[r2].at[:, :, pl.ds(0, 128)],
                            out_hbm.at[l, :, wid],
                            osem[r2],
                        )
            return carry

        lax.fori_loop(0, (LB + 1) // 2, pairbody, 0)
        for r2 in range(2):
            pltpu.make_async_copy(
                obuf[r2].at[:, :, pl.ds(0, 128)],
                out_hbm.at[0, :, wid],
                osem[r2],
            ).wait()

    return k(ids_p, table2, pe)


def kernel(input_ids, table):
    ids_p = jnp.transpose(
        input_ids.astype(jnp.int32).reshape(NW, 128, LB, 8), (2, 0, 3, 1)
    )
    table2 = table.reshape(VOCAB // 2, 2 * HIDDEN)
    out5 = _sc_embed(ids_p, table2, jnp.asarray(_PE))
    return jnp.transpose(out5, (2, 4, 0, 1, 3)).reshape(BATCH, SEQ, HIDDEN)


# speedup vs baseline: 1.8049x; 1.8049x over previous
import functools
import math

import jax
import jax.numpy as jnp
import numpy as np
from jax import lax
from jax.experimental import pallas as pl
from jax.experimental.pallas import tpu as pltpu
from jax.experimental.pallas import tpu_sc as plsc

VOCAB = 1000000
HIDDEN = 64
MAX_LEN = 512
BATCH = 4096
SEQ = 200

NC = 2
NS = 16
NW = NC * NS
LB = SEQ // 8
LANES = 16
PITCH = 133


def _make_pe_np(hidden_size=HIDDEN, max_len=MAX_LEN):
    position = np.arange(0, max_len, dtype=np.float32)[:, None]
    div_term = np.exp(
        np.arange(0, hidden_size, 2, dtype=np.float32)
        * (-math.log(10000.0) / hidden_size)
    )
    pe = np.zeros((max_len, hidden_size), dtype=np.float32)
    pe[:, 0::2] = np.sin(position * div_term)
    pe[:, 1::2] = np.cos(position * div_term)
    return pe


_PE = _make_pe_np()[:SEQ].reshape(-1)


def _sc_embed(ids_p, table2, pe):
    mesh = plsc.VectorSubcoreMesh(core_axis_name="c", subcore_axis_name="s")

    @functools.partial(
        pl.kernel,
        out_type=jax.ShapeDtypeStruct((SEQ, 8, NW, 8, 128), jnp.float32),
        mesh=mesh,
        compiler_params=pltpu.CompilerParams(
            use_tc_tiling_on_sc=False, needs_layout_passes=False
        ),
        scratch_types=(
            [pltpu.VMEM((SEQ * HIDDEN,), jnp.float32)]
            + [pltpu.VMEM((8, 128), jnp.int32)] * 2
            + [pltpu.VMEM((128, HIDDEN), jnp.float32)] * 4
            + [pltpu.VMEM((8, 8, PITCH), jnp.float32)] * 2
            + [pltpu.SemaphoreType.DMA] * 8
        ),
    )
    def k(ids_hbm, table_hbm, pe_hbm, out_hbm, pe_v, *rest):
        ibuf = rest[0:2]
        gbuf = rest[2:6]
        obuf = rest[6:8]
        isem = rest[8:10]
        gsem = rest[10:14]
        osem = rest[14:16]
        wid = lax.axis_index("s") * NC + lax.axis_index("c")

        pltpu.sync_copy(pe_hbm, pe_v)
        pltpu.sync_copy(ids_hbm.at[0, wid], ibuf[0])
        pltpu.async_copy(ids_hbm.at[1, wid], ibuf[1], isem[1])

        iota = lax.iota(jnp.int32, LANES)
        hvec = [iota + cg * LANES for cg in range(4)]
        i0c = [h // 8 for h in hvec]
        i1c = [h % 8 for h in hvec]

        pltpu.async_copy(table_hbm.at[ibuf[0].at[0]], gbuf[0], gsem[0])
        pltpu.async_copy(table_hbm.at[ibuf[0].at[1]], gbuf[1], gsem[1])

        def pairbody(pp, carry):
            for q in range(2):
                p = pp * 2 + q

                @pl.when(p < LB)
                def _():
                    for kq in range(8):
                        l = p * 8 + kq
                        g4 = kq % 4
                        r2 = kq % 2

                        if kq == 6:
                            @pl.when(p < LB - 1)
                            def _():
                                pltpu.make_async_copy(
                                    ids_hbm.at[0, wid], ibuf[1 - q],
                                    isem[1 - q],
                                ).wait()

                        @pl.when(l < SEQ - 2)
                        def _():
                            n4 = (kq + 2) % 4
                            nsl = ibuf[q] if kq < 6 else ibuf[1 - q]
                            pltpu.async_copy(
                                table_hbm.at[nsl.at[(kq + 2) % 8]],
                                gbuf[n4], gsem[n4],
                            )

                        pltpu.make_async_copy(
                            table_hbm.at[ibuf[q].at[kq]], gbuf[g4], gsem[g4]
                        ).wait()

                        if kq == 7:
                            @pl.when(p < LB - 2)
                            def _():
                                pltpu.async_copy(
                                    ids_hbm.at[p + 2, wid], ibuf[q], isem[q]
                                )

                        @pl.when(l >= 2)
                        def _():
                            pltpu.make_async_copy(
                                obuf[r2].at[:, :, pl.ds(0, 128)],
                                out_hbm.at[0, :, wid],
                                osem[r2],
                            ).wait()

                        pev = [
                            pe_v[pl.ds(l * HIDDEN + cg * LANES, LANES)]
                            for cg in range(4)
                        ]

                        @plsc.parallel_loop(0, 128, unroll=4)
                        def rowloop(r):
                            rb = jnp.broadcast_to(r, (LANES,))
                            for cg in range(4):
                                v = plsc.load_gather(
                                    gbuf[g4], [rb, hvec[cg]]
                                )
                                plsc.store_scatter(
                                    obuf[r2], [i0c[cg], i1c[cg], rb],
                                    v + pev[cg],
                                )

                        pltpu.async_copy(
                            obuf[r2].at[:, :, pl.ds(0, 128)],
                            out_hbm.at[l, :, wid],
                            osem[r2],
                        )
            return carry

        lax.fori_loop(0, (LB + 1) // 2, pairbody, 0)
        for r2 in range(2):
            pltpu.make_async_copy(
                obuf[r2].at[:, :, pl.ds(0, 128)],
                out_hbm.at[0, :, wid],
                osem[r2],
            ).wait()

    return k(ids_p, table2, pe)


def kernel(input_ids, table):
    ids_p = jnp.transpose(
        input_ids.astype(jnp.int32).reshape(NW, 128, LB, 8), (2, 0, 3, 1)
    )
    table2 = table
    out5 = _sc_embed(ids_p, table2, jnp.asarray(_PE))
    return jnp.transpose(out5, (2, 4, 0, 1, 3)).reshape(BATCH, SEQ, HIDDEN)
